# baseline (speedup 1.0000x reference)
import jax
import jax.numpy as jnp
from jax.experimental import pallas as pl
from jax.experimental.pallas import tpu as pltpu

_VOCABS = (40001, 30001, 28001)
_OFFSETS = (0, 40001, 70002)

_TB = 256


def _gather_body(tb):
    def body(idx_ref, table_ref, o_ref):
        base = pl.program_id(0) * tb
        for mi in range(tb):
            o_ref[mi, 0] = table_ref[idx_ref[base + mi], 0]
    return body


def kernel(table, user_id, item_id, cate_id):
    v, d = table.shape
    cols = [
        jnp.clip(raw.astype(jnp.int32), 0, vocab - 1) + off
        for raw, vocab, off in zip(
            (user_id, item_id, cate_id), _VOCABS, _OFFSETS)
    ]
    idx = jnp.stack(cols, axis=1).reshape(-1)
    n = idx.shape[0]
    table3 = table.reshape(v, 1, d)

    out = pl.pallas_call(
        _gather_body(_TB),
        out_shape=jax.ShapeDtypeStruct((n, 1, d), table.dtype),
        grid_spec=pltpu.PrefetchScalarGridSpec(
            num_scalar_prefetch=1,
            grid=(n // _TB,),
            in_specs=[pl.BlockSpec((v, 1, d), lambda i, idx_ref: (0, 0, 0))],
            out_specs=pl.BlockSpec((_TB, 1, d), lambda i, idx_ref: (i, 0, 0)),
        ),
        compiler_params=pltpu.CompilerParams(
            dimension_semantics=("parallel",),
        ),
    )(idx, table3)
    b = user_id.shape[0]
    return out.reshape(b, len(_VOCABS), d)


# speedup vs baseline: 2.7340x; 2.7340x over previous
import jax
import jax.numpy as jnp
from jax.experimental import pallas as pl
from jax.experimental.pallas import tpu as pltpu

_VOCABS = (40001, 30001, 28001)
_OFFSETS = (0, 40001, 70002)

_TB = 256


def _gather_body(tb):
    def body(idx_ref, table_ref, o_ref):
        base = pl.program_id(0) * tb
        for g in range(tb // 8):
            rows = []
            for j in range(8):
                r = idx_ref[base + 8 * g + j]
                b8 = pl.multiple_of((r >> 3) << 3, 8)
                chunk = table_ref[pl.ds(b8, 8), :]
                rows.append(pltpu.roll(chunk, -(r & 7), axis=0)[0:1, :])
            o_ref[pl.ds(8 * g, 8), :] = jnp.concatenate(rows, axis=0)
    return body


def kernel(table, user_id, item_id, cate_id):
    v, d = table.shape
    cols = [
        jnp.clip(raw.astype(jnp.int32), 0, vocab - 1) + off
        for raw, vocab, off in zip(
            (user_id, item_id, cate_id), _VOCABS, _OFFSETS)
    ]
    idx = jnp.stack(cols, axis=1).reshape(-1)
    n = idx.shape[0]

    out = pl.pallas_call(
        _gather_body(_TB),
        out_shape=jax.ShapeDtypeStruct((n, d), table.dtype),
        grid_spec=pltpu.PrefetchScalarGridSpec(
            num_scalar_prefetch=1,
            grid=(n // _TB,),
            in_specs=[pl.BlockSpec((v, d), lambda i, idx_ref: (0, 0))],
            out_specs=pl.BlockSpec((_TB, d), lambda i, idx_ref: (i, 0)),
        ),
        compiler_params=pltpu.CompilerParams(
            dimension_semantics=("parallel",),
        ),
    )(idx, table)
    b = user_id.shape[0]
    return out.reshape(b, len(_VOCABS), d)


# speedup vs baseline: 2.8391x; 1.0384x over previous
import jax
import jax.numpy as jnp
from jax.experimental import pallas as pl
from jax.experimental.pallas import tpu as pltpu

_VOCABS = (40001, 30001, 28001)
_OFFSETS = (0, 40001, 70002)

_TB = 256


def _gather_body(tb):
    def body(b8_ref, sh_ref, table_ref, o_ref):
        base = pl.program_id(0) * tb
        for g in range(tb // 8):
            rows = []
            for j in range(8):
                b8 = pl.multiple_of(b8_ref[base + 8 * g + j], 8)
                chunk = table_ref[pl.ds(b8, 8), :]
                rows.append(
                    pltpu.roll(chunk, sh_ref[base + 8 * g + j], axis=0)[0:1, :])
            o_ref[pl.ds(8 * g, 8), :] = jnp.concatenate(rows, axis=0)
    return body


def kernel(table, user_id, item_id, cate_id):
    v, d = table.shape
    cols = [
        jnp.clip(raw.astype(jnp.int32), 0, vocab - 1) + off
        for raw, vocab, off in zip(
            (user_id, item_id, cate_id), _VOCABS, _OFFSETS)
    ]
    idx = jnp.stack(cols, axis=1).reshape(-1)
    n = idx.shape[0]
    b8 = idx & ~7
    sh = (-idx) & 7

    out = pl.pallas_call(
        _gather_body(_TB),
        out_shape=jax.ShapeDtypeStruct((n, d), table.dtype),
        grid_spec=pltpu.PrefetchScalarGridSpec(
            num_scalar_prefetch=2,
            grid=(n // _TB,),
            in_specs=[pl.BlockSpec((v, d), lambda i, b8_ref, sh_ref: (0, 0))],
            out_specs=pl.BlockSpec((_TB, d), lambda i, b8_ref, sh_ref: (i, 0)),
        ),
        compiler_params=pltpu.CompilerParams(
            dimension_semantics=("parallel",),
        ),
    )(b8, sh, table)
    b = user_id.shape[0]
    return out.reshape(b, len(_VOCABS), d)
